# Initial kernel scaffold; baseline (speedup 1.0000x reference)
import jax
import jax.numpy as jnp
from jax.experimental import pallas as pl


def kernel(inputs, targets):
    raise NotImplementedError("write your pallas kernel here")


# speedup vs baseline: 1.0527x; 1.0527x over previous
import functools

import jax
import jax.numpy as jnp
from jax.experimental import pallas as pl
from jax.experimental.pallas import tpu as pltpu

GAMMA_ = 2.0
EPS_ = 1e-07


def _focal_block(x, t):
    y = jnp.where(t == 1, x, -x)
    q = jax.nn.sigmoid(y)
    om = 1.0 - q
    return jnp.sum(om * om * (-jnp.log(q + EPS_)))


def _fl_kernel(x_ref, t_ref, out_ref):
    i = pl.program_id(0)

    @pl.when(i == 0)
    def _():
        out_ref[0, 0] = 0.0

    out_ref[0, 0] += _focal_block(x_ref[...], t_ref[...])


def kernel(inputs, targets):
    n_rows, n_cols = inputs.shape
    block_rows = 512
    grid = n_rows // block_rows
    out = pl.pallas_call(
        _fl_kernel,
        grid=(grid,),
        in_specs=[
            pl.BlockSpec((block_rows, n_cols), lambda i: (i, 0)),
            pl.BlockSpec((block_rows, n_cols), lambda i: (i, 0)),
        ],
        out_specs=pl.BlockSpec(memory_space=pltpu.SMEM),
        out_shape=jax.ShapeDtypeStruct((1, 1), jnp.float32),
        compiler_params=pltpu.CompilerParams(
            dimension_semantics=("arbitrary",),
        ),
    )(inputs, targets)
    return out[0, 0] / (n_rows * n_cols)
